# baseline (speedup 1.0000x reference)
import functools
import jax
import jax.numpy as jnp
from jax import lax
from jax.experimental import pallas as pl
from jax.experimental.pallas import tpu as pltpu
from jax.experimental.pallas import tpu_sc as plsc


def _sc_weights16(v, last_cell):
    tr = v.astype(jnp.int32)
    trf = tr.astype(jnp.float32)
    idx = jnp.where(trf == v, tr - 1, tr)
    idx = jnp.minimum(jnp.maximum(idx, 0), last_cell)
    t = v - idx.astype(jnp.float32)
    t2 = t * t
    t3 = t2 * t
    h0 = 1.0 - 3.0 * t2 + 2.0 * t3
    h1 = t - 2.0 * t2 + t3
    h2 = 3.0 * t2 - 2.0 * t3
    h3 = t3 - t2
    return idx.astype(jnp.float32), h0, h1, h2, h3


def _sc_taps(xs, ys, n):
    q = xs.shape[0]
    info = plsc.get_sparse_core_info()
    nw = info.num_cores * info.num_subcores
    chunk = q // nw
    mesh = plsc.VectorSubcoreMesh(core_axis_name="c", subcore_axis_name="s")

    @functools.partial(
        pl.kernel, mesh=mesh,
        out_type=[jax.ShapeDtypeStruct((4 * q,), jnp.float32),
                  jax.ShapeDtypeStruct((6 * q,), jnp.float32)],
        scratch_types=[pltpu.VMEM((chunk,), jnp.float32)
                       for _ in range(12)],
    )
    def k(xs_hbm, ys_hbm, xw_hbm, yw_hbm,
          xv, yv, x0, x1, x2, x3, y0, y1, y2, y3, y4, y5):
        wid = lax.axis_index("s") * info.num_cores + lax.axis_index("c")
        base = wid * chunk
        pltpu.sync_copy(xs_hbm.at[pl.ds(base, chunk)], xv)
        pltpu.sync_copy(ys_hbm.at[pl.ds(base, chunk)], yv)
        for i in range(chunk // 16):
            sl = pl.ds(i * 16, 16)
            idx, h0, h1, h2, h3 = _sc_weights16(xv[sl], n - 2)
            lastx = idx == float(n - 2)
            x0[sl] = idx
            x1[sl] = jnp.where(lastx, h0 - h1 - h3, h0 - h1)
            x2[sl] = jnp.where(lastx, h1 + h2 + h3, h1 + h2 - h3)
            x3[sl] = jnp.where(lastx, 0.0, h3)
            idy, g0, g1, g2, g3 = _sc_weights16(yv[sl], n - 2)
            lasty = idy == float(n - 2)
            y0[sl] = idy
            y1[sl] = g0
            y2[sl] = g2
            y3[sl] = jnp.where(lasty, -(g1 + g3), -g1)
            y4[sl] = jnp.where(lasty, g1 + g3, g1 - g3)
            y5[sl] = jnp.where(lasty, 0.0, g3)
        for j, ref in enumerate((x0, x1, x2, x3)):
            pltpu.sync_copy(ref, xw_hbm.at[pl.ds(j * q + base, chunk)])
        for j, ref in enumerate((y0, y1, y2, y3, y4, y5)):
            pltpu.sync_copy(ref, yw_hbm.at[pl.ds(j * q + base, chunk)])

    return k(xs, ys)


def _three_tap_f(pos, idx, c0, c1, c2):
    zero = jnp.zeros((), jnp.float32)
    return (jnp.where(pos == idx, c0, zero)
            + jnp.where(pos == idx + 1, c1, zero)
            + jnp.where(pos == idx + 2, c2, zero))


def _tc_body(xw_ref, ywt_ref, sig_ref, out_ref, bx_ref, ayt_ref, cyt_ref):
    b = pl.program_id(0)
    n = sig_ref.shape[1]
    q = xw_ref.shape[1]

    @pl.when(b == 0)
    def _build_bases():
        rows = jax.lax.broadcasted_iota(jnp.int32, (n, q), 0)
        bx_ref[...] = _three_tap_f(rows, xw_ref[0:1, :].astype(jnp.int32),
                                   xw_ref[1:2, :],
                                   xw_ref[2:3, :], xw_ref[3:4, :])
        cols = jax.lax.broadcasted_iota(jnp.int32, (q, n), 1)
        iy = ywt_ref[:, 0:1].astype(jnp.int32)
        ayt_ref[...] = _three_tap_f(cols, iy, ywt_ref[:, 1:2],
                                    ywt_ref[:, 2:3],
                                    jnp.zeros_like(ywt_ref[:, 0:1]))
        cyt_ref[...] = _three_tap_f(cols, iy, ywt_ref[:, 3:4],
                                    ywt_ref[:, 4:5], ywt_ref[:, 5:6])

    s = sig_ref[0]
    t = jnp.dot(s, bx_ref[...], preferred_element_type=jnp.float32)
    out_ref[0] = (jnp.dot(ayt_ref[...], t, preferred_element_type=jnp.float32)
                  + jnp.dot(cyt_ref[...], s, preferred_element_type=jnp.float32))


def kernel(xs, ys, xaxis, yaxis, signal):
    del xaxis, yaxis
    b, n, _ = signal.shape
    q = xs.shape[0]
    xw, yw = _sc_taps(xs, ys, n)
    xw4 = xw.reshape(4, q)
    ywt = yw.reshape(6, q).T
    return pl.pallas_call(
        _tc_body,
        grid=(b,),
        in_specs=[
            pl.BlockSpec((4, q), lambda i: (0, 0)),
            pl.BlockSpec((q, 6), lambda i: (0, 0)),
            pl.BlockSpec((1, n, n), lambda i: (i, 0, 0)),
        ],
        out_specs=pl.BlockSpec((1, q, q), lambda i: (i, 0, 0)),
        out_shape=jax.ShapeDtypeStruct((b, q, q), jnp.float32),
        scratch_shapes=[
            pltpu.VMEM((n, q), jnp.float32),
            pltpu.VMEM((q, n), jnp.float32),
            pltpu.VMEM((q, n), jnp.float32),
        ],
    )(xw4, ywt, signal)


# speedup vs baseline: 1.3441x; 1.3441x over previous
import jax
import jax.numpy as jnp
from jax.experimental import pallas as pl
from jax.experimental.pallas import tpu as pltpu


def _hermite(v, n):
    idx = jnp.clip(jnp.ceil(v).astype(jnp.int32) - 1, 0, n - 2)
    t = v - idx.astype(v.dtype)
    t2 = t * t
    t3 = t2 * t
    h0 = 1.0 - 3.0 * t2 + 2.0 * t3
    h1 = t - 2.0 * t2 + t3
    h2 = 3.0 * t2 - 2.0 * t3
    h3 = t3 - t2
    return idx, h0, h1, h2, h3


def _three_tap(pos, idx, c0, c1, c2):
    zero = jnp.zeros((), jnp.float32)
    return (jnp.where(pos == idx, c0, zero)
            + jnp.where(pos == idx + 1, c1, zero)
            + jnp.where(pos == idx + 2, c2, zero))


def _body(xs_ref, ys_ref, sig_ref, out_ref, bx_ref, d_ref):
    b = pl.program_id(0)
    n = sig_ref.shape[1]
    q = xs_ref.shape[1]

    @pl.when(b == 0)
    def _build_bases():
        ix, h0, h1, h2, h3 = _hermite(xs_ref[...], n)
        last = ix == n - 2
        w0 = jnp.where(last, h0 - h1 - h3, h0 - h1)
        w1 = jnp.where(last, h1 + h2 + h3, h1 + h2 - h3)
        w2 = jnp.where(last, 0.0, h3)
        rows = jax.lax.broadcasted_iota(jnp.int32, (n, q), 0)
        bx_ref[...] = _three_tap(rows, ix, w0, w1, w2)

        iy, g0, g1, g2, g3 = _hermite(ys_ref[...], n)
        lasty = iy == n - 2
        c0 = jnp.where(lasty, -(g1 + g3), -g1)
        c1 = jnp.where(lasty, g1 + g3, g1 - g3)
        c2 = jnp.where(lasty, 0.0, g3)
        cols = jax.lax.broadcasted_iota(jnp.int32, (q, n), 1)
        d_ref[0:q, :] = _three_tap(cols, iy, g0, g2, jnp.zeros_like(g0))
        d_ref[q:, :] = _three_tap(cols, iy, c0, c1, c2)

    v = jnp.dot(d_ref[...], sig_ref[0], preferred_element_type=jnp.float32)
    out_ref[0] = (jnp.dot(v[:q], bx_ref[...], preferred_element_type=jnp.float32)
                  + v[q:])


def kernel(xs, ys, xaxis, yaxis, signal):
    del xaxis, yaxis
    b, n, _ = signal.shape
    q = xs.shape[0]
    xs2 = xs.reshape(1, q)
    ys2 = ys.reshape(q, 1)
    return pl.pallas_call(
        _body,
        grid=(b,),
        in_specs=[
            pl.BlockSpec((1, q), lambda i: (0, 0)),
            pl.BlockSpec((q, 1), lambda i: (0, 0)),
            pl.BlockSpec((1, n, n), lambda i: (i, 0, 0)),
        ],
        out_specs=pl.BlockSpec((1, q, q), lambda i: (i, 0, 0)),
        out_shape=jax.ShapeDtypeStruct((b, q, q), jnp.float32),
        scratch_shapes=[
            pltpu.VMEM((n, q), jnp.float32),
            pltpu.VMEM((2 * q, n), jnp.float32),
        ],
    )(xs2, ys2, signal)
